# baseline (speedup 1.0000x reference)
import functools

import jax
import jax.numpy as jnp
from jax import lax
from jax.experimental import pallas as pl
from jax.experimental.pallas import tpu as pltpu
from jax.experimental.pallas import tpu_sc as plsc

NUM_ENT = 100000
NUM_REL = 116
DIM = 128
B = 16384

LANES = 16
CHUNK = 128


def _make_sc_comp(rows):
    info = plsc.get_sparse_core_info()
    nc, ns = info.num_cores, info.num_subcores
    nw = nc * ns
    b_per_w = rows // nw
    n_chunks = b_per_w // CHUNK
    mesh = plsc.VectorSubcoreMesh(core_axis_name="c", subcore_axis_name="s")

    @functools.partial(
        pl.kernel,
        mesh=mesh,
        out_type=jax.ShapeDtypeStruct((rows, DIM), jnp.float32),
        scratch_types=[
            pltpu.VMEM((n_chunks, CHUNK), jnp.int32),
            pltpu.VMEM((n_chunks, CHUNK), jnp.int32),
            pltpu.VMEM((2, CHUNK, DIM), jnp.float32),
            pltpu.VMEM((2, CHUNK, DIM), jnp.float32),
            pltpu.SemaphoreType.DMA,
            pltpu.SemaphoreType.DMA,
            pltpu.SemaphoreType.DMA,
            pltpu.SemaphoreType.DMA,
        ],
    )
    def sc_comp(src_hbm, dst_hbm, ent_hbm, out_hbm,
                si_v, di_v, h_v, t_v, g0, g1, o0, o1):
        wid = lax.axis_index("s") * nc + lax.axis_index("c")
        wc0 = wid * n_chunks
        base = wid * b_per_w
        gsem = (g0, g1)
        osem = (o0, o1)

        pltpu.sync_copy(src_hbm.at[pl.ds(wc0, n_chunks)], si_v)
        pltpu.sync_copy(dst_hbm.at[pl.ds(wc0, n_chunks)], di_v)

        def start_gather(j, s):
            sem = gsem[s]
            return (
                pltpu.async_copy(ent_hbm.at[si_v.at[j]], h_v.at[s], sem),
                pltpu.async_copy(ent_hbm.at[di_v.at[j]], t_v.at[s], sem),
            )

        gathers = [None, None]
        outs = [None, None]
        gathers[0] = start_gather(0, 0)
        for j in range(n_chunks):
            s = j & 1
            if j + 1 < n_chunks:
                if outs[1 - s] is not None:
                    outs[1 - s].wait()
                    outs[1 - s] = None
                gathers[1 - s] = start_gather(j + 1, 1 - s)
            for c in gathers[s]:
                c.wait()

            def row_body(i, carry, s=s):
                for c in range(DIM // LANES):
                    sl = pl.ds(c * LANES, LANES)
                    h_v[s, i, sl] = h_v[s, i, sl] - t_v[s, i, sl]
                return carry

            lax.fori_loop(0, CHUNK, row_body, 0)
            outs[s] = pltpu.async_copy(
                h_v.at[s], out_hbm.at[pl.ds(base + j * CHUNK, CHUNK)], osem[s])
        for s in range(2):
            if outs[s] is not None:
                outs[s].wait()

    return sc_comp


_sc_comp = _make_sc_comp(B)


def _tc_body(comp_ref, rel_ref, w_ref, relt_ref, b_ref, out_ref):
    acc = jnp.dot(comp_ref[...], w_ref[...], preferred_element_type=jnp.float32)
    wr = jnp.dot(relt_ref[...], w_ref[...],
                 preferred_element_type=jnp.float32) + b_ref[...]
    rel_ids = rel_ref[0, 0, :]
    onehot = (rel_ids[:, None] == lax.broadcasted_iota(
        jnp.int32, (rel_ids.shape[0], DIM), 1)).astype(jnp.float32)
    racc = jnp.dot(onehot, wr, preferred_element_type=jnp.float32)
    out_ref[...] = jnp.tanh(acc + racc)


def kernel(src, rel, dst, ent_table, rel_table, W, b):
    nrc = B // CHUNK
    comp = _sc_comp(src.reshape(nrc, CHUNK), dst.reshape(nrc, CHUNK), ent_table)
    blk = 1024
    nblk = B // blk
    rel3 = rel.reshape(nblk, 1, blk)
    relt_pad = jnp.zeros((DIM, DIM), jnp.float32).at[:NUM_REL].set(rel_table)
    out = pl.pallas_call(
        _tc_body,
        grid=(nblk,),
        in_specs=[
            pl.BlockSpec((blk, DIM), lambda i: (i, 0)),
            pl.BlockSpec((1, 1, blk), lambda i: (i, 0, 0)),
            pl.BlockSpec((DIM, DIM), lambda i: (0, 0)),
            pl.BlockSpec((DIM, DIM), lambda i: (0, 0)),
            pl.BlockSpec((1, DIM), lambda i: (0, 0)),
        ],
        out_specs=pl.BlockSpec((blk, DIM), lambda i: (i, 0)),
        out_shape=jax.ShapeDtypeStruct((B, DIM), jnp.float32),
    )(comp, rel3, W, relt_pad, b.reshape(1, DIM))
    return out


# speedup vs baseline: 1.1661x; 1.1661x over previous
import functools

import jax
import jax.numpy as jnp
from jax import lax
from jax.experimental import pallas as pl
from jax.experimental.pallas import tpu as pltpu
from jax.experimental.pallas import tpu_sc as plsc

NUM_ENT = 100000
NUM_REL = 116
DIM = 128
B = 16384

LANES = 16
CHUNK = 128


def _make_sc_comp(rows):
    info = plsc.get_sparse_core_info()
    nc, ns = info.num_cores, info.num_subcores
    nw = nc * ns
    b_per_w = rows // nw
    n_chunks = b_per_w // CHUNK
    mesh = plsc.VectorSubcoreMesh(core_axis_name="c", subcore_axis_name="s")

    @functools.partial(
        pl.kernel,
        mesh=mesh,
        out_type=jax.ShapeDtypeStruct((rows, DIM), jnp.float32),
        scratch_types=[
            pltpu.VMEM((n_chunks, CHUNK), jnp.int32),
            pltpu.VMEM((n_chunks, CHUNK), jnp.int32),
            pltpu.VMEM((2, CHUNK, DIM), jnp.float32),
            pltpu.VMEM((2, CHUNK, DIM), jnp.float32),
            pltpu.SemaphoreType.DMA,
            pltpu.SemaphoreType.DMA,
            pltpu.SemaphoreType.DMA,
            pltpu.SemaphoreType.DMA,
        ],
    )
    def sc_comp(src_hbm, dst_hbm, ent_hbm, out_hbm,
                si_v, di_v, h_v, t_v, g0, g1, o0, o1):
        wid = lax.axis_index("s") * nc + lax.axis_index("c")
        wc0 = wid * n_chunks
        base = wid * b_per_w
        gsem = (g0, g1)
        osem = (o0, o1)

        pltpu.sync_copy(src_hbm.at[pl.ds(wc0, n_chunks)], si_v)
        pltpu.sync_copy(dst_hbm.at[pl.ds(wc0, n_chunks)], di_v)

        def start_gather(j, s):
            sem = gsem[s]
            return (
                pltpu.async_copy(ent_hbm.at[si_v.at[j]], h_v.at[s], sem),
                pltpu.async_copy(ent_hbm.at[di_v.at[j]], t_v.at[s], sem),
            )

        gathers = [None, None]
        outs = [None, None]
        gathers[0] = start_gather(0, 0)
        for j in range(n_chunks):
            s = j & 1
            if j + 1 < n_chunks:
                if outs[1 - s] is not None:
                    outs[1 - s].wait()
                    outs[1 - s] = None
                gathers[1 - s] = start_gather(j + 1, 1 - s)
            for c in gathers[s]:
                c.wait()

            def row_body(i, carry, s=s):
                for c in range(DIM // LANES):
                    sl = pl.ds(c * LANES, LANES)
                    h_v[s, i, sl] = h_v[s, i, sl] - t_v[s, i, sl]
                return carry

            lax.fori_loop(0, CHUNK, row_body, 0)
            outs[s] = pltpu.async_copy(
                h_v.at[s], out_hbm.at[pl.ds(base + j * CHUNK, CHUNK)], osem[s])
        for s in range(2):
            if outs[s] is not None:
                outs[s].wait()

    return sc_comp


_sc_comp = _make_sc_comp(B)


def _tc_body(comp_ref, rel_ref, w_ref, relt_ref, b_ref, out_ref):
    acc = jnp.dot(comp_ref[...], w_ref[...], preferred_element_type=jnp.float32)
    wr = jnp.dot(relt_ref[...], w_ref[...],
                 preferred_element_type=jnp.float32) + b_ref[...]
    rel_ids = rel_ref[0, 0, :]
    onehot = (rel_ids[:, None] == lax.broadcasted_iota(
        jnp.int32, (rel_ids.shape[0], DIM), 1)).astype(jnp.float32)
    racc = jnp.dot(onehot, wr, preferred_element_type=jnp.float32)
    out_ref[...] = jnp.tanh(acc + racc)


def kernel(src, rel, dst, ent_table, rel_table, W, b):
    nrc = B // CHUNK
    comp = _sc_comp(src.reshape(nrc, CHUNK), dst.reshape(nrc, CHUNK), ent_table)
    blk = 4096
    nblk = B // blk
    rel3 = rel.reshape(nblk, 1, blk)
    relt_pad = jnp.zeros((DIM, DIM), jnp.float32).at[:NUM_REL].set(rel_table)
    out = pl.pallas_call(
        _tc_body,
        grid=(nblk,),
        in_specs=[
            pl.BlockSpec((blk, DIM), lambda i: (i, 0)),
            pl.BlockSpec((1, 1, blk), lambda i: (i, 0, 0)),
            pl.BlockSpec((DIM, DIM), lambda i: (0, 0)),
            pl.BlockSpec((DIM, DIM), lambda i: (0, 0)),
            pl.BlockSpec((1, DIM), lambda i: (0, 0)),
        ],
        out_specs=pl.BlockSpec((blk, DIM), lambda i: (i, 0)),
        out_shape=jax.ShapeDtypeStruct((B, DIM), jnp.float32),
    )(comp, rel3, W, relt_pad, b.reshape(1, DIM))
    return out


# speedup vs baseline: 1.2195x; 1.0458x over previous
import functools

import jax
import jax.numpy as jnp
from jax import lax
from jax.experimental import pallas as pl
from jax.experimental.pallas import tpu as pltpu
from jax.experimental.pallas import tpu_sc as plsc

NUM_ENT = 100000
NUM_REL = 116
DIM = 128
B = 16384

LANES = 16
CHUNK = 128


def _make_sc_comp(rows):
    info = plsc.get_sparse_core_info()
    nc, ns = info.num_cores, info.num_subcores
    nw = nc * ns
    b_per_w = rows // nw
    n_chunks = b_per_w // CHUNK
    mesh = plsc.VectorSubcoreMesh(core_axis_name="c", subcore_axis_name="s")

    @functools.partial(
        pl.kernel,
        mesh=mesh,
        out_type=jax.ShapeDtypeStruct((rows, DIM), jnp.float32),
        scratch_types=[
            pltpu.VMEM((n_chunks, CHUNK), jnp.int32),
            pltpu.VMEM((n_chunks, CHUNK), jnp.int32),
            pltpu.VMEM((2, CHUNK, DIM), jnp.float32),
            pltpu.VMEM((2, CHUNK, DIM), jnp.float32),
            pltpu.SemaphoreType.DMA,
            pltpu.SemaphoreType.DMA,
            pltpu.SemaphoreType.DMA,
            pltpu.SemaphoreType.DMA,
        ],
    )
    def sc_comp(src_hbm, dst_hbm, ent_hbm, out_hbm,
                si_v, di_v, h_v, t_v, g0, g1, o0, o1):
        wid = lax.axis_index("s") * nc + lax.axis_index("c")
        wc0 = wid * n_chunks
        base = wid * b_per_w
        gsem = (g0, g1)
        osem = (o0, o1)

        pltpu.sync_copy(src_hbm.at[pl.ds(wc0, n_chunks)], si_v)
        pltpu.sync_copy(dst_hbm.at[pl.ds(wc0, n_chunks)], di_v)

        def start_gather(j, s):
            sem = gsem[s]
            return (
                pltpu.async_copy(ent_hbm.at[si_v.at[j]], h_v.at[s], sem),
                pltpu.async_copy(ent_hbm.at[di_v.at[j]], t_v.at[s], sem),
            )

        gathers = [None, None]
        outs = [None, None]
        gathers[0] = start_gather(0, 0)
        for j in range(n_chunks):
            s = j & 1
            if j + 1 < n_chunks:
                if outs[1 - s] is not None:
                    outs[1 - s].wait()
                    outs[1 - s] = None
                gathers[1 - s] = start_gather(j + 1, 1 - s)
            for c in gathers[s]:
                c.wait()

            def row_body(i, carry, s=s):
                for c in range(DIM // LANES):
                    sl = pl.ds(c * LANES, LANES)
                    h_v[s, i, sl] = h_v[s, i, sl] - t_v[s, i, sl]
                return carry

            lax.fori_loop(0, CHUNK, row_body, 0)
            outs[s] = pltpu.async_copy(
                h_v.at[s], out_hbm.at[pl.ds(base + j * CHUNK, CHUNK)], osem[s])
        for s in range(2):
            if outs[s] is not None:
                outs[s].wait()

    return sc_comp


_sc_comp = _make_sc_comp(B)


def _tc_body(comp_ref, rel_ref, w_ref, relt_ref, b_ref, out_ref):
    acc = jnp.dot(comp_ref[...], w_ref[...], preferred_element_type=jnp.float32)
    wr = jnp.dot(relt_ref[...], w_ref[...],
                 preferred_element_type=jnp.float32) + b_ref[...]
    rel_ids = rel_ref[0, 0, :]
    onehot = (rel_ids[:, None] == lax.broadcasted_iota(
        jnp.int32, (rel_ids.shape[0], DIM), 1)).astype(jnp.float32)
    racc = jnp.dot(onehot, wr, preferred_element_type=jnp.float32)
    out_ref[...] = jnp.tanh(acc + racc)


def kernel(src, rel, dst, ent_table, rel_table, W, b):
    nrc = B // CHUNK
    comp = _sc_comp(src.reshape(nrc, CHUNK), dst.reshape(nrc, CHUNK), ent_table)
    blk = 8192
    nblk = B // blk
    rel3 = rel.reshape(nblk, 1, blk)
    relt_pad = jnp.zeros((DIM, DIM), jnp.float32).at[:NUM_REL].set(rel_table)
    out = pl.pallas_call(
        _tc_body,
        grid=(nblk,),
        in_specs=[
            pl.BlockSpec((blk, DIM), lambda i: (i, 0)),
            pl.BlockSpec((1, 1, blk), lambda i: (i, 0, 0)),
            pl.BlockSpec((DIM, DIM), lambda i: (0, 0)),
            pl.BlockSpec((DIM, DIM), lambda i: (0, 0)),
            pl.BlockSpec((1, DIM), lambda i: (0, 0)),
        ],
        out_specs=pl.BlockSpec((blk, DIM), lambda i: (i, 0)),
        out_shape=jax.ShapeDtypeStruct((B, DIM), jnp.float32),
    )(comp, rel3, W, relt_pad, b.reshape(1, DIM))
    return out
